# baseline (speedup 1.0000x reference)
import functools

import jax
import jax.numpy as jnp
from jax import lax
from jax.experimental import pallas as pl
from jax.experimental.pallas import tpu as pltpu
from jax.experimental.pallas import tpu_sc as plsc

NUM_BUCKETS = 100000
EMB = 32
BATCH = 16384
MAX_LEN = 100
PLEN = 104

NC = 2
NS = 16
NW = NC * NS
ROWS_PER_W = BATCH // NW
G = 4
NCHUNK = ROWS_PER_W // G
RPC = G * PLEN // 32
I32R_PER_W = ROWS_PER_W * PLEN // 32
BF16R_PER_W = 2 * I32R_PER_W
HALF = BF16R_PER_W // 2
NBUF = 2


def _body(big_hbm, out_hbm, stage_bf, idx2, rows_v, out_v, *sems):
    wid = lax.axis_index("s") * NC + lax.axis_index("c")
    base = wid * ROWS_PER_W

    for h in range(2):
        pltpu.sync_copy(
            big_hbm.at[pl.ds(NUM_BUCKETS + wid * BF16R_PER_W + h * HALF, HALF)],
            stage_bf,
        )

        def cvt(r2, _):
            w0 = plsc.bitcast(stage_bf[2 * r2, :], jnp.int32)
            w1 = plsc.bitcast(stage_bf[2 * r2 + 1, :], jnp.int32)
            v0 = (w0 & 0x1FFF) | (((w0 >> 16) & 0xF) << 13)
            v1 = (w1 & 0x1FFF) | (((w1 >> 16) & 0xF) << 13)
            idx2[h * (HALF // 2) + r2, pl.ds(0, 16)] = v0
            idx2[h * (HALF // 2) + r2, pl.ds(16, 16)] = v1
            return 0

        lax.fori_loop(0, HALF // 2, cvt, 0)

    iota = lax.iota(jnp.int32, 16)
    idx_even = iota * 2
    idx_odd = idx_even + 1
    scale = jnp.float32(1.0 / MAX_LEN)

    def start(c, b):
        for k in range(RPC):
            pltpu.async_copy(
                big_hbm.at[idx2.at[c * RPC + k, :]],
                rows_v.at[b, pl.ds(k * 32, 32)],
                sems[b],
            )

    def wait(c, b):
        for k in range(RPC):
            pltpu.make_async_copy(
                big_hbm.at[idx2.at[c * RPC + k, :]],
                rows_v.at[b, pl.ds(k * 32, 32)],
                sems[b],
            ).wait()

    def reduce_chunk(c, b):
        for g in range(G):
            acc_e = [jnp.zeros((16,), jnp.float32) for _ in range(4)]
            acc_o = [jnp.zeros((16,), jnp.float32) for _ in range(4)]
            for j in range(PLEN):
                row = rows_v[b, g * PLEN + j, :]
                e, o = plsc.unpack(
                    row,
                    format=plsc.PackFormat.INTERLEAVED,
                    preferred_element_type=jnp.float32,
                )
                acc_e[j % 4] += e
                acc_o[j % 4] += o
            s_e = ((acc_e[0] + acc_e[1]) + (acc_e[2] + acc_e[3])) * scale
            s_o = ((acc_o[0] + acc_o[1]) + (acc_o[2] + acc_o[3])) * scale
            r = c * G + g
            plsc.store_scatter(out_v.at[r], [idx_even], s_e)
            plsc.store_scatter(out_v.at[r], [idx_odd], s_o)

    for b in range(NBUF):
        start(b, b)

    def loop_body(i, _):
        c = i * NBUF
        for b in range(NBUF):
            wait(c + b, b)
            reduce_chunk(c + b, b)
            start(c + b + NBUF, b)
        return 0

    lax.fori_loop(0, NCHUNK // NBUF - 1, loop_body, 0)

    c_last = NCHUNK - NBUF
    for b in range(NBUF):
        wait(c_last + b, b)
        reduce_chunk(c_last + b, b)

    pltpu.sync_copy(out_v, out_hbm.at[pl.ds(base, ROWS_PER_W)])


@functools.partial(jax.jit, donate_argnums=())
def _run(big):
    mesh = plsc.VectorSubcoreMesh(
        core_axis_name="c", subcore_axis_name="s", num_cores=NC, num_subcores=NS
    )
    f = pl.kernel(
        _body,
        out_type=jax.ShapeDtypeStruct((BATCH, EMB), jnp.float32),
        mesh=mesh,
        scratch_types=[
            pltpu.VMEM((HALF, EMB), jnp.bfloat16),
            pltpu.VMEM((I32R_PER_W, 32), jnp.int32),
            pltpu.VMEM((NBUF, G * PLEN, EMB), jnp.bfloat16),
            pltpu.VMEM((ROWS_PER_W, EMB), jnp.float32),
        ]
        + [pltpu.SemaphoreType.DMA] * NBUF,
        compiler_params=pltpu.CompilerParams(
            use_tc_tiling_on_sc=False, needs_layout_passes=False
        ),
    )
    return f(big)


def kernel(input, embed_weight):
    table_bf = embed_weight.astype(jnp.bfloat16)
    idp = jnp.pad(input, ((0, 0), (0, PLEN - MAX_LEN))).reshape(-1)
    lo = ((idp & 0x1FFF) | 0x4000).astype(jnp.uint16)
    hi = (((idp >> 13) & 0xF) | 0x4000).astype(jnp.uint16)
    enc = jnp.stack([lo, hi], axis=-1).reshape(-1)
    inp_rows = jax.lax.bitcast_convert_type(enc, jnp.bfloat16).reshape(-1, EMB)
    big = jnp.concatenate([table_bf, inp_rows], axis=0)
    return _run(big)


# speedup vs baseline: 8.7764x; 8.7764x over previous
import functools

import jax
import jax.numpy as jnp
from jax import lax
from jax.experimental import pallas as pl
from jax.experimental.pallas import tpu as pltpu
from jax.experimental.pallas import tpu_sc as plsc

NUM_BUCKETS = 100000
EMB = 32
BATCH = 16384
MAX_LEN = 100

NC = 2
NS = 16
NW = NC * NS
ROWS_PER_W = BATCH // NW
NBUF = 4


def _body(table_hbm, inp_hbm, out_hbm, idx_slab, rows_v, out_v, *sems):
    wid = lax.axis_index("s") * NC + lax.axis_index("c")
    base = wid * ROWS_PER_W

    pltpu.sync_copy(inp_hbm.at[pl.ds(base, ROWS_PER_W)], idx_slab)

    def start(r, b):
        pltpu.async_copy(table_hbm.at[idx_slab.at[r]], rows_v.at[b], sems[b])

    def wait(r, b):
        pltpu.make_async_copy(
            table_hbm.at[idx_slab.at[r]], rows_v.at[b], sems[b]
        ).wait()

    def reduce_row(r, b):
        for h in range(2):
            accs = [jnp.zeros((16,), jnp.float32) for _ in range(4)]
            for j in range(MAX_LEN):
                accs[j % 4] += rows_v[b, j, pl.ds(h * 16, 16)]
            s = (accs[0] + accs[1]) + (accs[2] + accs[3])
            out_v[r, pl.ds(h * 16, 16)] = s * jnp.float32(1.0 / MAX_LEN)

    for b in range(NBUF):
        start(b, b)

    def loop_body(i, _):
        r = i * NBUF
        for b in range(NBUF):
            wait(r + b, b)
            reduce_row(r + b, b)
            start(r + b + NBUF, b)
        return 0

    lax.fori_loop(0, ROWS_PER_W // NBUF - 1, loop_body, 0)

    r_last = ROWS_PER_W - NBUF
    for b in range(NBUF):
        wait(r_last + b, b)
        reduce_row(r_last + b, b)

    pltpu.sync_copy(out_v, out_hbm.at[pl.ds(base, ROWS_PER_W)])


@functools.partial(jax.jit, donate_argnums=())
def _run(table, inp):
    mesh = plsc.VectorSubcoreMesh(
        core_axis_name="c", subcore_axis_name="s", num_cores=NC, num_subcores=NS
    )
    f = pl.kernel(
        _body,
        out_type=jax.ShapeDtypeStruct((BATCH, EMB), jnp.float32),
        mesh=mesh,
        scratch_types=[
            pltpu.VMEM((ROWS_PER_W, MAX_LEN), jnp.int32),
            pltpu.VMEM((NBUF, MAX_LEN, EMB), jnp.float32),
            pltpu.VMEM((ROWS_PER_W, EMB), jnp.float32),
        ]
        + [pltpu.SemaphoreType.DMA] * NBUF,
        compiler_params=pltpu.CompilerParams(use_tc_tiling_on_sc=False),
    )
    return f(table, inp)


def kernel(input, embed_weight):
    return _run(embed_weight, input)
